# baseline (speedup 1.0000x reference)
import functools

import jax
import jax.numpy as jnp
from jax import lax
from jax.experimental import pallas as pl
from jax.experimental.pallas import tpu as pltpu
from jax.experimental.pallas import tpu_sc as plsc

NC = 2
NS = 16
NW = NC * NS
LANES = 16


def _edge_kernels(n_nodes, epad, full_groups, epw):
    N = n_nodes
    TBL = N * N
    ROW = LANES
    PADS = epad - epw
    PGROUPS = PADS // LANES
    NCHUNK = 4
    CH = epad // NCHUNK
    CHG = CH // LANES

    mesh = plsc.VectorSubcoreMesh(core_axis_name="c", subcore_axis_name="s")

    def fill_keys(es_v, ed_v, kref, g_lo, g_hi, scale):
        def g_body(g, carry):
            a = es_v[pl.ds(g * LANES, LANES)]
            b = ed_v[pl.ds(g * LANES, LANES)]
            lo = jnp.minimum(a, b)
            hi = jnp.maximum(a, b)
            kref[pl.ds((g - g_lo) * LANES, LANES)] = (lo * N + hi) * scale
            return carry

        lax.fori_loop(g_lo, g_hi, g_body, None)

    def fill_pad_keys(wid, kref, g_lo, scale):
        iota = lax.iota(jnp.int32, LANES)
        for j in range(PGROUPS):
            col = (full_groups + j - g_lo) * LANES
            p0 = wid * PADS + j * LANES + 1
            kref[pl.ds(col, LANES)] = ((iota + p0) * N) * scale

    @functools.partial(
        pl.kernel,
        out_type=jax.ShapeDtypeStruct((TBL, ROW), jnp.int32),
        mesh=mesh,
        scratch_types=[
            pltpu.VMEM((epad,), jnp.int32),
            pltpu.VMEM((epad,), jnp.int32),
            [pltpu.VMEM((CH,), jnp.int32) for _ in range(NCHUNK)],
            [pltpu.VMEM((CH, ROW), jnp.int32) for _ in range(2)],
            [pltpu.SemaphoreType.DMA for _ in range(2)],
        ],
        compiler_params=pltpu.CompilerParams(
            needs_layout_passes=False, use_tc_tiling_on_sc=False),
    )
    def scatter_ids(src_hbm, dst_hbm, table_out,
                    es_v, ed_v, keys, wides, sems):
        c = lax.axis_index("c")
        s = lax.axis_index("s")
        wid = s * NC + c
        pltpu.sync_copy(src_hbm.at[pl.ds(wid * epw, epw)],
                        es_v.at[pl.ds(0, epw)])
        pltpu.sync_copy(dst_hbm.at[pl.ds(wid * epw, epw)],
                        ed_v.at[pl.ds(0, epw)])

        iota = lax.iota(jnp.int32, LANES)
        zcol = jnp.zeros((LANES,), jnp.int32)
        base_id = wid * epad

        def id_fill(wide_v, chunk_base):
            def body(g, carry):
                ids = base_id + chunk_base + g * LANES + iota
                plsc.store_scatter(wide_v, [g * LANES + iota, zcol], ids)
                return carry
            lax.fori_loop(0, CHG, body, None)

        pend = [None, None]
        for k in range(NCHUNK):
            g_lo = k * CHG
            g_hi = min((k + 1) * CHG, full_groups)
            fill_keys(es_v, ed_v, keys[k], g_lo, g_hi, 1)
            if g_hi < (k + 1) * CHG:
                fill_pad_keys(wid, keys[k], g_lo, 1)
            b = k % 2
            if pend[b] is not None:
                pend[b].wait()
            id_fill(wides[b], k * CH)
            pend[b] = pltpu.async_copy(wides[b], table_out.at[keys[k]],
                                       sems[b])
        pend[0].wait()
        pend[1].wait()

    NPAD = ((N + 10 * LANES * NS - 1) // (10 * LANES * NS)) * (10 * LANES * NS)

    @functools.partial(
        pl.kernel,
        out_type=jax.ShapeDtypeStruct((NW, NPAD), jnp.float32),
        mesh=mesh,
        scratch_types=[
            pltpu.VMEM((epad,), jnp.int32),
            pltpu.VMEM((epad,), jnp.int32),
            [pltpu.VMEM((CH,), jnp.int32) for _ in range(NCHUNK)],
            pltpu.VMEM((epad,), jnp.int32),
            pltpu.VMEM((NPAD,), jnp.float32),
            [pltpu.SemaphoreType.DMA for _ in range(NCHUNK)],
        ],
        compiler_params=pltpu.CompilerParams(needs_layout_passes=False),
    )
    def count_winners(table_hbm, src_hbm, dst_hbm, deg_out,
                      es_v, ed_v, keys, got_v, deg_l, sems):
        c = lax.axis_index("c")
        s = lax.axis_index("s")
        wid = s * NC + c
        pltpu.sync_copy(src_hbm.at[pl.ds(wid * epw, epw)],
                        es_v.at[pl.ds(0, epw)])
        pltpu.sync_copy(dst_hbm.at[pl.ds(wid * epw, epw)],
                        ed_v.at[pl.ds(0, epw)])

        gathers = []
        for k in range(NCHUNK):
            g_lo = k * CHG
            g_hi = min((k + 1) * CHG, full_groups)
            fill_keys(es_v, ed_v, keys[k], g_lo, g_hi, LANES)
            if g_hi < (k + 1) * CHG:
                fill_pad_keys(wid, keys[k], g_lo, LANES)
            gathers.append(pltpu.async_copy(
                table_hbm.at[keys[k]], got_v.at[pl.ds(k * CH, CH)], sems[k]))

        zero = jnp.zeros((LANES,), jnp.float32)

        def z_body(i, carry):
            deg_l[pl.ds(i * LANES, LANES)] = zero
            return carry

        lax.fori_loop(0, NPAD // LANES, z_body, None)

        iota = lax.iota(jnp.int32, LANES)
        base_id = wid * epad
        one = jnp.full((LANES,), 1.0, jnp.float32)

        def c_body(g, carry):
            sl = pl.ds(g * LANES, LANES)
            myid = base_id + g * LANES + iota
            w = jnp.where(got_v[sl] == myid, one, zero)
            plsc.addupdate_scatter(deg_l, [es_v[sl]], w)
            plsc.addupdate_scatter(deg_l, [ed_v[sl]], w)
            return carry

        for k in range(NCHUNK):
            gathers[k].wait()
            g_lo = k * CHG
            g_hi = min((k + 1) * CHG, full_groups)
            lax.fori_loop(g_lo, g_hi, c_body, None)
        pltpu.sync_copy(deg_l, deg_out.at[wid])

    return scatter_ids, count_winners, NPAD


def _dense_kernel(x_ref, dp_ref, w1_ref, w2_ref, b1_ref, b2_ref,
                  t1_ref, t2_ref, o_ref):
    deg = jnp.sum(dp_ref[...], axis=0)
    sd = jnp.sqrt(deg)
    c1 = jnp.cos(sd * t1_ref[0, 0])[:, None]
    c2 = jnp.cos(sd * t2_ref[0, 0])[:, None]
    h = lax.dot_general(x_ref[...], w1_ref[...], (((1,), (1,)), ((), ())),
                        preferred_element_type=jnp.float32)
    h = c1 * h + b1_ref[...]
    h = jnp.maximum(h, 0.0)
    h = lax.dot_general(h, w2_ref[...], (((1,), (1,)), ((), ())),
                        preferred_element_type=jnp.float32)
    h = c2 * h + b2_ref[...]
    m = jnp.max(h, axis=1, keepdims=True)
    ex = jnp.exp(h - m)
    sm = jnp.sum(ex, axis=1, keepdims=True)
    o_ref[...] = h - m - jnp.log(sm)


def kernel(x, edge_index, W1, b1, t1, W2, b2, t2):
    n = x.shape[0]
    d_in = x.shape[1]
    d_out = W2.shape[0]
    e = edge_index.shape[1]

    epw = e // NW
    assert epw * NW == e and epw % LANES == 0
    epad = ((epw + 127) // 128) * 128
    if epad == epw:
        epad += 128
    full_groups = epw // LANES

    ei = edge_index.astype(jnp.int32)

    scatter_ids, count_winners, npad = _edge_kernels(n, epad, full_groups, epw)
    table = scatter_ids(ei[0], ei[1])
    deg_parts = count_winners(jnp.reshape(table, (-1,)), ei[0], ei[1])

    rb = npad // 8
    grid = (npad // rb,)
    out = pl.pallas_call(
        _dense_kernel,
        grid=grid,
        in_specs=[
            pl.BlockSpec((rb, d_in), lambda i: (i, 0)),
            pl.BlockSpec((NW, rb), lambda i: (0, i)),
            pl.BlockSpec(W1.shape, lambda i: (0, 0)),
            pl.BlockSpec(W2.shape, lambda i: (0, 0)),
            pl.BlockSpec((1, d_in), lambda i: (0, 0)),
            pl.BlockSpec((1, d_out), lambda i: (0, 0)),
            pl.BlockSpec((1, 1), lambda i: (0, 0)),
            pl.BlockSpec((1, 1), lambda i: (0, 0)),
        ],
        out_specs=pl.BlockSpec((rb, d_out), lambda i: (i, 0)),
        out_shape=jax.ShapeDtypeStruct((n, d_out), jnp.float32),
    )(x, deg_parts, W1, W2, b1.reshape(1, -1), b2.reshape(1, -1),
      jnp.reshape(t1, (1, 1)), jnp.reshape(t2, (1, 1)))
    return out


# speedup vs baseline: 1.2285x; 1.2285x over previous
import functools

import jax
import jax.numpy as jnp
from jax import lax
from jax.experimental import pallas as pl
from jax.experimental.pallas import tpu as pltpu
from jax.experimental.pallas import tpu_sc as plsc

NC = 2
NS = 16
NW = NC * NS
LANES = 16


def _edge_kernels(n_nodes, epad, full_groups, epw):
    N = n_nodes
    TBL = N * N
    ROW = LANES
    PADS = epad - epw
    PGROUPS = PADS // LANES
    NCHUNK = 4
    CH = epad // NCHUNK
    CHG = CH // LANES

    mesh = plsc.VectorSubcoreMesh(core_axis_name="c", subcore_axis_name="s")

    def fill_keys(es_v, ed_v, kref, g_lo, g_hi, scale):
        def g_body(g, carry):
            a = es_v[pl.ds(g * LANES, LANES)]
            b = ed_v[pl.ds(g * LANES, LANES)]
            lo = jnp.minimum(a, b)
            hi = jnp.maximum(a, b)
            kref[pl.ds((g - g_lo) * LANES, LANES)] = (lo * N + hi) * scale
            return carry

        lax.fori_loop(g_lo, g_hi, g_body, None)

    def fill_pad_keys(wid, kref, g_lo, scale):
        iota = lax.iota(jnp.int32, LANES)
        for j in range(PGROUPS):
            col = (full_groups + j - g_lo) * LANES
            p0 = wid * PADS + j * LANES + 1
            kref[pl.ds(col, LANES)] = ((iota + p0) * N) * scale

    @functools.partial(
        pl.kernel,
        out_type=jax.ShapeDtypeStruct((TBL, ROW), jnp.int32),
        mesh=mesh,
        scratch_types=[
            pltpu.VMEM((epad,), jnp.int32),
            pltpu.VMEM((epad,), jnp.int32),
            [pltpu.VMEM((CH,), jnp.int32) for _ in range(NCHUNK)],
            [pltpu.VMEM((CH, ROW), jnp.int32) for _ in range(2)],
            [pltpu.SemaphoreType.DMA for _ in range(2)],
        ],
        compiler_params=pltpu.CompilerParams(
            needs_layout_passes=False, use_tc_tiling_on_sc=False),
    )
    def scatter_ids(edge_hbm, table_out,
                    es_v, ed_v, keys, wides, sems):
        c = lax.axis_index("c")
        s = lax.axis_index("s")
        wid = s * NC + c
        pltpu.sync_copy(edge_hbm.at[pl.ds(wid * epw, epw)],
                        es_v.at[pl.ds(0, epw)])
        pltpu.sync_copy(edge_hbm.at[pl.ds(NW * epw + wid * epw, epw)],
                        ed_v.at[pl.ds(0, epw)])

        iota = lax.iota(jnp.int32, LANES)
        zcol = jnp.zeros((LANES,), jnp.int32)
        base_id = wid * epad

        def id_fill(wide_v, chunk_base):
            def body(g, carry):
                ids = base_id + chunk_base + g * LANES + iota
                plsc.store_scatter(wide_v, [g * LANES + iota, zcol], ids)
                return carry
            lax.fori_loop(0, CHG, body, None)

        pend = [None, None]
        for k in range(NCHUNK):
            g_lo = k * CHG
            g_hi = min((k + 1) * CHG, full_groups)
            fill_keys(es_v, ed_v, keys[k], g_lo, g_hi, 1)
            if g_hi < (k + 1) * CHG:
                fill_pad_keys(wid, keys[k], g_lo, 1)
            b = k % 2
            if pend[b] is not None:
                pend[b].wait()
            id_fill(wides[b], k * CH)
            pend[b] = pltpu.async_copy(wides[b], table_out.at[keys[k]],
                                       sems[b])
        pend[0].wait()
        pend[1].wait()

    NPAD = ((N + 10 * LANES * NS - 1) // (10 * LANES * NS)) * (10 * LANES * NS)

    @functools.partial(
        pl.kernel,
        out_type=jax.ShapeDtypeStruct((NW, NPAD), jnp.float32),
        mesh=mesh,
        scratch_types=[
            pltpu.VMEM((epad,), jnp.int32),
            pltpu.VMEM((epad,), jnp.int32),
            [pltpu.VMEM((CH,), jnp.int32) for _ in range(NCHUNK)],
            pltpu.VMEM((epad,), jnp.int32),
            pltpu.VMEM((NPAD,), jnp.float32),
            [pltpu.SemaphoreType.DMA for _ in range(NCHUNK)],
        ],
        compiler_params=pltpu.CompilerParams(needs_layout_passes=False),
    )
    def count_winners(table_hbm, edge_hbm, deg_out,
                      es_v, ed_v, keys, got_v, deg_l, sems):
        c = lax.axis_index("c")
        s = lax.axis_index("s")
        wid = s * NC + c
        pltpu.sync_copy(edge_hbm.at[pl.ds(wid * epw, epw)],
                        es_v.at[pl.ds(0, epw)])
        pltpu.sync_copy(edge_hbm.at[pl.ds(NW * epw + wid * epw, epw)],
                        ed_v.at[pl.ds(0, epw)])

        gathers = []
        for k in range(NCHUNK):
            g_lo = k * CHG
            g_hi = min((k + 1) * CHG, full_groups)
            fill_keys(es_v, ed_v, keys[k], g_lo, g_hi, LANES)
            if g_hi < (k + 1) * CHG:
                fill_pad_keys(wid, keys[k], g_lo, LANES)
            gathers.append(pltpu.async_copy(
                table_hbm.at[keys[k]], got_v.at[pl.ds(k * CH, CH)], sems[k]))

        zero = jnp.zeros((LANES,), jnp.float32)

        def z_body(i, carry):
            deg_l[pl.ds(i * LANES, LANES)] = zero
            return carry

        lax.fori_loop(0, NPAD // LANES, z_body, None)

        iota = lax.iota(jnp.int32, LANES)
        base_id = wid * epad
        one = jnp.full((LANES,), 1.0, jnp.float32)

        for k in range(NCHUNK):
            gathers[k].wait()
            g_lo = k * CHG
            g_hi = min((k + 1) * CHG, full_groups)

            @functools.partial(plsc.parallel_loop, g_lo, g_hi, unroll=4)
            def _(g):
                sl = pl.ds(g * LANES, LANES)
                myid = base_id + g * LANES + iota
                w = jnp.where(got_v[sl] == myid, one, zero)
                plsc.addupdate_scatter(deg_l, [es_v[sl]], w)
                plsc.addupdate_scatter(deg_l, [ed_v[sl]], w)
        pltpu.sync_copy(deg_l, deg_out.at[wid])

    return scatter_ids, count_winners, NPAD


def _dense_kernel(x_ref, dp_ref, w1_ref, w2_ref, b1_ref, b2_ref,
                  t1_ref, t2_ref, o_ref):
    deg = jnp.sum(dp_ref[...], axis=0)
    sd = jnp.sqrt(deg)
    c1 = jnp.cos(sd * t1_ref[0, 0])[:, None]
    c2 = jnp.cos(sd * t2_ref[0, 0])[:, None]
    h = lax.dot_general(x_ref[...], w1_ref[...], (((1,), (1,)), ((), ())),
                        preferred_element_type=jnp.float32)
    h = c1 * h + b1_ref[...]
    h = jnp.maximum(h, 0.0)
    h = lax.dot_general(h, w2_ref[...], (((1,), (1,)), ((), ())),
                        preferred_element_type=jnp.float32)
    h = c2 * h + b2_ref[...]
    m = jnp.max(h, axis=1, keepdims=True)
    ex = jnp.exp(h - m)
    sm = jnp.sum(ex, axis=1, keepdims=True)
    o_ref[...] = h - m - jnp.log(sm)


def kernel(x, edge_index, W1, b1, t1, W2, b2, t2):
    n = x.shape[0]
    d_in = x.shape[1]
    d_out = W2.shape[0]
    e = edge_index.shape[1]

    epw = e // NW
    assert epw * NW == e and epw % LANES == 0
    epad = ((epw + 127) // 128) * 128
    if epad == epw:
        epad += 128
    full_groups = epw // LANES

    ei = edge_index.astype(jnp.int32).reshape(-1)

    scatter_ids, count_winners, npad = _edge_kernels(n, epad, full_groups, epw)
    table = scatter_ids(ei)
    deg_parts = count_winners(jnp.reshape(table, (-1,)), ei)

    rb = npad // 8
    grid = (npad // rb,)
    out = pl.pallas_call(
        _dense_kernel,
        grid=grid,
        in_specs=[
            pl.BlockSpec((rb, d_in), lambda i: (i, 0)),
            pl.BlockSpec((NW, rb), lambda i: (0, i)),
            pl.BlockSpec(W1.shape, lambda i: (0, 0)),
            pl.BlockSpec(W2.shape, lambda i: (0, 0)),
            pl.BlockSpec((1, d_in), lambda i: (0, 0)),
            pl.BlockSpec((1, d_out), lambda i: (0, 0)),
            pl.BlockSpec((1, 1), lambda i: (0, 0)),
            pl.BlockSpec((1, 1), lambda i: (0, 0)),
        ],
        out_specs=pl.BlockSpec((rb, d_out), lambda i: (i, 0)),
        out_shape=jax.ShapeDtypeStruct((n, d_out), jnp.float32),
    )(x, deg_parts, W1, W2, b1.reshape(1, -1), b2.reshape(1, -1),
      jnp.reshape(t1, (1, 1)), jnp.reshape(t2, (1, 1)))
    return out
